# Initial kernel scaffold; baseline (speedup 1.0000x reference)
import jax
import jax.numpy as jnp
from jax.experimental import pallas as pl


def kernel(T_k, T_q, relative_attention_bias):
    raise NotImplementedError("write your pallas kernel here")


# speedup vs baseline: 42.0259x; 42.0259x over previous
import functools
import math

import jax
import jax.numpy as jnp
from jax import lax
from jax.experimental import pallas as pl
from jax.experimental.pallas import tpu as pltpu
from jax.experimental.pallas import tpu_sc as plsc

_NUM_HEADS = 16
_NUM_BUCKETS = 32
_MAX_DISTANCE = 128
_T = 2048
_LINE = 4096
_NSHIFT = 8
_FLAT = _NSHIFT * _LINE
_NC = 2
_NS = 16
_ROWS_PER_W = _NUM_HEADS * _T // (_NC * _NS)
_K = 8


def _line_tc_kernel(delta_ref, table_ref, line_ref):
    p = lax.broadcasted_iota(jnp.int32, (_NUM_BUCKETS, _FLAT), 1)
    m = p >> 12
    d = p & (_LINE - 1)
    rel = d + m - (_T - 1) + delta_ref[0]
    nb = _NUM_BUCKETS // 2
    rb = jnp.where(rel > 0, nb, 0)
    a = jnp.abs(rel)
    max_exact = nb // 2
    is_small = a < max_exact
    large = max_exact + (
        jnp.log(a.astype(jnp.float32) / max_exact)
        / math.log(_MAX_DISTANCE / max_exact)
        * (nb - max_exact)
    ).astype(jnp.int32)
    large = jnp.minimum(large, nb - 1)
    bucket = rb + jnp.where(is_small, a, large)
    b_iota = lax.broadcasted_iota(jnp.int32, (_NUM_BUCKETS, _FLAT), 0)
    onehot = (b_iota == bucket).astype(jnp.float32)
    line_ref[...] = jnp.dot(table_ref[...], onehot,
                            preferred_element_type=jnp.float32,
                            precision=lax.Precision.HIGHEST)


def _compute_line(delta, table):
    return pl.pallas_call(
        _line_tc_kernel,
        out_shape=jax.ShapeDtypeStruct((_NUM_HEADS, _FLAT), jnp.float32),
        in_specs=[
            pl.BlockSpec(memory_space=pltpu.SMEM),
            pl.BlockSpec(memory_space=pltpu.VMEM),
        ],
        out_specs=pl.BlockSpec(memory_space=pltpu.VMEM),
    )(delta, table)


def _expand_sc(line_flat):
    mesh = plsc.VectorSubcoreMesh(core_axis_name="c", subcore_axis_name="s")

    @functools.partial(
        pl.kernel,
        mesh=mesh,
        out_type=jax.ShapeDtypeStruct((_NUM_HEADS * _T * _T,), jnp.float32),
        scratch_types=[
            pltpu.VMEM((_FLAT,), jnp.float32),
            pltpu.SemaphoreType.DMA,
        ],
    )
    def k(line_hbm, out_hbm, line_v, sem):
        wid = lax.axis_index("s") * _NC + lax.axis_index("c")
        h = wid // (_T // _ROWS_PER_W)
        row0 = (wid % (_T // _ROWS_PER_W)) * _ROWS_PER_W
        pltpu.sync_copy(
            line_hbm.at[pl.ds(pl.multiple_of(h * _FLAT, _NSHIFT), _FLAT)],
            line_v)

        def chunk(t, carry):
            cps = []
            for j in range(_K):
                r = row0 + t * _K + j
                off = (_T - 1) - r
                mm = lax.rem(off, _NSHIFT)
                src = pl.multiple_of(
                    mm * _LINE + (off - mm), _NSHIFT)
                dst = pl.multiple_of((h * _T + r) * _T, _NSHIFT)
                cp = pltpu.make_async_copy(
                    line_v.at[pl.ds(src, _T)],
                    out_hbm.at[pl.ds(dst, _T)], sem)
                cp.start()
                cps.append(cp)
            for cp in cps:
                cp.wait()
            return carry

        lax.fori_loop(0, _ROWS_PER_W // _K, chunk, 0)

    return k(line_flat)


def kernel(T_k, T_q, relative_attention_bias):
    delta = (jnp.asarray(T_k, jnp.int32)
             - jnp.asarray(T_q, jnp.int32)).reshape(1)
    line = _compute_line(delta, relative_attention_bias)
    out_flat = _expand_sc(jnp.reshape(line, (-1,)))
    return jnp.reshape(out_flat, (_NUM_HEADS, _T, _T))
